# baseline (speedup 1.0000x reference)
import functools

import jax
import jax.numpy as jnp
from jax import lax
from jax.experimental import pallas as pl
from jax.experimental.pallas import tpu as pltpu
from jax.experimental.pallas import tpu_sc as plsc

N = 100000
E = 1600000
B = 512
NW = 32
NODE_PER = 3136
NODE_TOT = 100352
EDGE_PER = E // NW
EBLK = 2000
CL = 16384


GD = (NODE_TOT + CL - 1) // CL
GE = (E + CL - 1) // CL


def _lane_triples(pd, rd, md, po, ro, mo):
    def emit(p_ref, r_ref, m_ref, base, n_valid, outs):
        d = p_ref[...].reshape(16, CL) - r_ref[...].reshape(16, CL)
        mm = m_ref[...].reshape(16, CL)
        a = jnp.abs(d)
        t = a * mm
        ok = base + lax.broadcasted_iota(jnp.int32, (1, CL), 1) < n_valid
        ones = jnp.full((1, 16), 1.0, jnp.float32)
        red = lambda x: jnp.dot(ones, x, preferred_element_type=jnp.float32)
        vals = (red(t), red(t * a), red(mm))
        for o_ref, v in zip(outs, vals):
            o_ref[...] = jnp.where(ok, v, 0.0).reshape(CL)

    def body(pd_ref, rd_ref, md_ref, po_ref, ro_ref, mo_ref,
             maed_ref, msed_ref, numd_ref, maeo_ref, mseo_ref, numo_ref):
        i = pl.program_id(0)

        @pl.when(i < GD)
        def _():
            emit(pd_ref, rd_ref, md_ref, i * CL, N,
                 (maed_ref, msed_ref, numd_ref))

        @pl.when(i >= GD)
        def _():
            emit(po_ref, ro_ref, mo_ref, (i - GD) * CL, E,
                 (maeo_ref, mseo_ref, numo_ref))

    dspec = pl.BlockSpec((4, 4, CL), lambda i: (0, 0, jnp.minimum(i, GD - 1)))
    ospec = pl.BlockSpec((4, 4, CL), lambda i: (0, 0, jnp.maximum(i - GD, 0)))
    dout = pl.BlockSpec((CL,), lambda i: (jnp.minimum(i, GD - 1),))
    oout = pl.BlockSpec((CL,), lambda i: (jnp.maximum(i - GD, 0),))
    return pl.pallas_call(
        body,
        grid=(GD + GE,),
        in_specs=[dspec] * 3 + [ospec] * 3,
        out_specs=[dout] * 3 + [oout] * 3,
        out_shape=[jax.ShapeDtypeStruct((NODE_TOT,), jnp.float32)] * 3
        + [jax.ShapeDtypeStruct((E,), jnp.float32)] * 3,
    )(pd, rd, md, po, ro, mo)


def _segment_partials(batch_pad, maed, msed, numd, dst, maeo, mseo, numo):
    mesh = plsc.VectorSubcoreMesh(core_axis_name="c", subcore_axis_name="s")
    nblk = EDGE_PER // EBLK
    NB = NODE_PER // 2

    @functools.partial(
        pl.kernel,
        mesh=mesh,
        compiler_params=pltpu.CompilerParams(needs_layout_passes=False),
        out_type=jax.ShapeDtypeStruct((NW, 6, 512), jnp.float32),
        scratch_types=[
            pltpu.VMEM((N,), jnp.int32),
            pltpu.VMEM((EBLK,), jnp.int32),
            pltpu.VMEM((EBLK,), jnp.float32),
            pltpu.VMEM((EBLK,), jnp.float32),
            pltpu.VMEM((EBLK,), jnp.float32),
            pltpu.VMEM((EBLK,), jnp.int32),
            pltpu.VMEM((EBLK,), jnp.float32),
            pltpu.VMEM((EBLK,), jnp.float32),
            pltpu.VMEM((EBLK,), jnp.float32),
            pltpu.SemaphoreType.DMA,
            pltpu.SemaphoreType.DMA,
            pltpu.SemaphoreType.DMA,
        ] + [pltpu.VMEM((B,), jnp.float32)] * 12,
    )
    def sc_kernel(batch_hbm, maed_hbm, msed_hbm, numd_hbm,
                  dst_hbm, maeo_hbm, mseo_hbm, numo_hbm, out_hbm,
                  table_v, i0_v, a0_v, b0_v, c0_v, i1_v, a1_v, b1_v, c1_v,
                  sem0, sem1, semt,
                  d0A, d1A, d2A, o0A, o1A, o2A,
                  d0B, d1B, d2B, o0B, o1B, o2B):
        wid = lax.axis_index("s") * 2 + lax.axis_index("c")
        accsA = (d0A, d1A, d2A, o0A, o1A, o2A)
        accsB = (d0B, d1B, d2B, o0B, o1B, o2B)
        sems = (sem0, sem1)
        bufsets = ((i0_v, a0_v, b0_v, c0_v), (i1_v, a1_v, b1_v, c1_v))

        th = pltpu.make_async_copy(batch_hbm.at[pl.ds(0, N)], table_v, semt)
        th.start()

        def zero_step(i, _):
            z = jnp.zeros((16,), jnp.float32)
            for a in accsA + accsB:
                a[pl.ds(i * 16, 16)] = z
            return 0
        lax.fori_loop(0, B // 16, zero_step, 0)

        def start(srcs, off, ln, k):
            hs = []
            for hb, v in zip(srcs, bufsets[k]):
                h = pltpu.make_async_copy(hb.at[pl.ds(off, ln)],
                                          v.at[pl.ds(0, ln)], sems[k])
                h.start()
                hs.append(h)
            return hs

        def scat(acc3, seg, k, i):
            for a, v in zip(acc3, bufsets[k][1:]):
                plsc.addupdate_scatter(a, [seg], v[pl.ds(i * 16, 16)])

        nsrc = (batch_hbm, maed_hbm, msed_hbm, numd_hbm)
        nbase = wid * NODE_PER
        h0 = start(nsrc, nbase, NB, 0)
        h1 = start(nsrc, nbase + NB, NB, 1)
        for k, hs in ((0, h0), (1, h1)):
            for h in hs:
                h.wait()

            def node_pair(j, _, k=k):
                for t, acc3 in ((0, accsA[0:3]), (1, accsB[0:3])):
                    i = 2 * j + t
                    seg = bufsets[k][0][pl.ds(i * 16, 16)]
                    scat(acc3, seg, k, i)
                return 0
            lax.fori_loop(0, NB // 32, node_pair, 0, unroll=4)

        th.wait()
        esrc = (dst_hbm, maeo_hbm, mseo_hbm, numo_hbm)
        ebase = wid * EDGE_PER
        pend = start(esrc, ebase, EBLK, 0)
        for b in range(nblk):
            k = b % 2
            for h in pend:
                h.wait()
            if b + 1 < nblk:
                pend = start(esrc, ebase + (b + 1) * EBLK, EBLK, (b + 1) % 2)

            def edge_pair(j, _, k=k):
                for t, acc3 in ((0, accsA[3:6]), (1, accsB[3:6])):
                    i = 2 * j + t
                    idx = bufsets[k][0][pl.ds(i * 16, 16)]
                    seg = plsc.load_gather(table_v, [idx])
                    scat(acc3, seg, k, i)
                return 0
            lax.fori_loop(0, EBLK // 32, edge_pair, 0, unroll=4)

            i_tail = EBLK // 16 - 1
            idx = bufsets[k][0][pl.ds(i_tail * 16, 16)]
            seg = plsc.load_gather(table_v, [idx])
            scat(accsA[3:6], seg, k, i_tail)

        def merge_step(i, _):
            for aA, aB in zip(accsA, accsB):
                aA[pl.ds(i * 16, 16)] = (aA[pl.ds(i * 16, 16)]
                                         + aB[pl.ds(i * 16, 16)])
            return 0
        lax.fori_loop(0, B // 16, merge_step, 0)

        for q, a in enumerate(accsA):
            pltpu.sync_copy(a, out_hbm.at[wid, q])

    return sc_kernel(batch_pad, maed, msed, numd, dst, maeo, mseo, numo)


def _combine(partials, bi_last):
    def body(p_ref, bi_ref, out_ref):
        s = jnp.sum(p_ref[...], axis=0)
        maed, msed, numd = s[0:1], s[1:2], s[2:3]
        maeo, mseo, numo = s[3:4], s[4:5], s[5:6]
        nseg_i = bi_ref[0, 0] + 1
        msk = lax.broadcasted_iota(jnp.int32, (1, B), 1) < nseg_i
        nseg = nseg_i.astype(jnp.float32)
        num = numd + numo
        batch_mae = jnp.sum(jnp.where(msk, (maed + maeo) / num, 0.0)) / nseg
        batch_mse = jnp.sum(jnp.where(msk, (msed + mseo) / num, 0.0)) / nseg
        batch_loss = batch_mae + jnp.sqrt(batch_mse)
        diag_mae = jnp.sum(jnp.where(msk, maed / numd, 0.0)) / nseg
        off_mae = jnp.sum(jnp.where(msk, maeo / numo, 0.0)) / nseg
        lane = lax.broadcasted_iota(jnp.int32, (1, 128), 1)
        out_ref[...] = (jnp.where(lane == 0, batch_loss, 0.0)
                        + jnp.where(lane == 1, batch_mae, 0.0)
                        + jnp.where(lane == 2, diag_mae, 0.0)
                        + jnp.where(lane == 3, off_mae, 0.0))

    return pl.pallas_call(
        body,
        in_specs=[pl.BlockSpec(memory_space=pltpu.VMEM),
                  pl.BlockSpec(memory_space=pltpu.SMEM)],
        out_specs=pl.BlockSpec(memory_space=pltpu.VMEM),
        out_shape=jax.ShapeDtypeStruct((1, 128), jnp.float32),
    )(partials, bi_last)


def kernel(pred_diag, pred_off_diag, real_diag, real_off_diag,
           mask_diag, mask_off_diag, batch_index, dst_index):
    t = lambda x: x.transpose(1, 2, 0)
    maed, msed, numd, maeo, mseo, numo = _lane_triples(
        t(pred_diag), t(real_diag), t(mask_diag),
        t(pred_off_diag), t(real_off_diag), t(mask_off_diag))

    batch_pad = jnp.pad(batch_index.astype(jnp.int32), (0, NODE_TOT - N))

    partials = _segment_partials(batch_pad, maed, msed, numd,
                                 dst_index.astype(jnp.int32),
                                 maeo, mseo, numo)

    bi_last = batch_index[N - 1:].astype(jnp.int32).reshape(1, 1)
    out = _combine(partials, bi_last)
    return (out[0, 0], out[0, 1], out[0, 2], out[0, 3])


# speedup vs baseline: 1.1278x; 1.1278x over previous
import functools

import jax
import jax.numpy as jnp
from jax import lax
from jax.experimental import pallas as pl
from jax.experimental.pallas import tpu as pltpu
from jax.experimental.pallas import tpu_sc as plsc

N = 100000
E = 1600000
B = 512
NW = 32
NODE_PER = 3136
NODE_TOT = 100352
EDGE_PER = E // NW
EBLK = 2000
CL = 32768


GD = (NODE_TOT + CL - 1) // CL
GE = (E + CL - 1) // CL


def _lane_triples(pd, rd, md, po, ro, mo):
    def emit(p_ref, r_ref, m_ref, base, n_valid, outs):
        d = p_ref[...].reshape(16, CL) - r_ref[...].reshape(16, CL)
        mm = m_ref[...].reshape(16, CL)
        a = jnp.abs(d)
        t = a * mm
        ok = base + lax.broadcasted_iota(jnp.int32, (1, CL), 1) < n_valid
        ones = jnp.full((1, 16), 1.0, jnp.float32)
        red = lambda x: jnp.dot(ones, x, preferred_element_type=jnp.float32)
        vals = (red(t), red(t * a), red(mm))
        for o_ref, v in zip(outs, vals):
            o_ref[...] = jnp.where(ok, v, 0.0).reshape(CL)

    def body(pd_ref, rd_ref, md_ref, po_ref, ro_ref, mo_ref,
             maed_ref, msed_ref, numd_ref, maeo_ref, mseo_ref, numo_ref):
        i = pl.program_id(0)

        @pl.when(i < GD)
        def _():
            emit(pd_ref, rd_ref, md_ref, i * CL, N,
                 (maed_ref, msed_ref, numd_ref))

        @pl.when(i >= GD)
        def _():
            emit(po_ref, ro_ref, mo_ref, (i - GD) * CL, E,
                 (maeo_ref, mseo_ref, numo_ref))

    dspec = pl.BlockSpec((4, 4, CL), lambda i: (0, 0, jnp.minimum(i, GD - 1)))
    ospec = pl.BlockSpec((4, 4, CL), lambda i: (0, 0, jnp.maximum(i - GD, 0)))
    dout = pl.BlockSpec((CL,), lambda i: (jnp.minimum(i, GD - 1),))
    oout = pl.BlockSpec((CL,), lambda i: (jnp.maximum(i - GD, 0),))
    return pl.pallas_call(
        body,
        grid=(GD + GE,),
        in_specs=[dspec] * 3 + [ospec] * 3,
        out_specs=[dout] * 3 + [oout] * 3,
        out_shape=[jax.ShapeDtypeStruct((NODE_TOT,), jnp.float32)] * 3
        + [jax.ShapeDtypeStruct((E,), jnp.float32)] * 3,
    )(pd, rd, md, po, ro, mo)


def _segment_partials(batch_pad, maed, msed, numd, dst, maeo, mseo, numo):
    mesh = plsc.VectorSubcoreMesh(core_axis_name="c", subcore_axis_name="s")
    nblk = EDGE_PER // EBLK
    NB = NODE_PER // 2

    @functools.partial(
        pl.kernel,
        mesh=mesh,
        compiler_params=pltpu.CompilerParams(needs_layout_passes=False),
        out_type=jax.ShapeDtypeStruct((NW, 6, 512), jnp.float32),
        scratch_types=[
            pltpu.VMEM((N,), jnp.int32),
            pltpu.VMEM((EBLK,), jnp.int32),
            pltpu.VMEM((EBLK,), jnp.float32),
            pltpu.VMEM((EBLK,), jnp.float32),
            pltpu.VMEM((EBLK,), jnp.float32),
            pltpu.VMEM((EBLK,), jnp.int32),
            pltpu.VMEM((EBLK,), jnp.float32),
            pltpu.VMEM((EBLK,), jnp.float32),
            pltpu.VMEM((EBLK,), jnp.float32),
            pltpu.SemaphoreType.DMA,
            pltpu.SemaphoreType.DMA,
            pltpu.SemaphoreType.DMA,
        ] + [pltpu.VMEM((B,), jnp.float32)] * 12,
    )
    def sc_kernel(batch_hbm, maed_hbm, msed_hbm, numd_hbm,
                  dst_hbm, maeo_hbm, mseo_hbm, numo_hbm, out_hbm,
                  table_v, i0_v, a0_v, b0_v, c0_v, i1_v, a1_v, b1_v, c1_v,
                  sem0, sem1, semt,
                  d0A, d1A, d2A, o0A, o1A, o2A,
                  d0B, d1B, d2B, o0B, o1B, o2B):
        wid = lax.axis_index("s") * 2 + lax.axis_index("c")
        accsA = (d0A, d1A, d2A, o0A, o1A, o2A)
        accsB = (d0B, d1B, d2B, o0B, o1B, o2B)
        sems = (sem0, sem1)
        bufsets = ((i0_v, a0_v, b0_v, c0_v), (i1_v, a1_v, b1_v, c1_v))

        th = pltpu.make_async_copy(batch_hbm.at[pl.ds(0, N)], table_v, semt)
        th.start()

        def zero_step(i, _):
            z = jnp.zeros((16,), jnp.float32)
            for a in accsA + accsB:
                a[pl.ds(i * 16, 16)] = z
            return 0
        lax.fori_loop(0, B // 16, zero_step, 0)

        def start(srcs, off, ln, k):
            hs = []
            for hb, v in zip(srcs, bufsets[k]):
                h = pltpu.make_async_copy(hb.at[pl.ds(off, ln)],
                                          v.at[pl.ds(0, ln)], sems[k])
                h.start()
                hs.append(h)
            return hs

        def scat(acc3, seg, k, i):
            for a, v in zip(acc3, bufsets[k][1:]):
                plsc.addupdate_scatter(a, [seg], v[pl.ds(i * 16, 16)])

        nsrc = (batch_hbm, maed_hbm, msed_hbm, numd_hbm)
        nbase = wid * NODE_PER
        h0 = start(nsrc, nbase, NB, 0)
        h1 = start(nsrc, nbase + NB, NB, 1)
        for k, hs in ((0, h0), (1, h1)):
            for h in hs:
                h.wait()

            def node_pair(j, _, k=k):
                for t, acc3 in ((0, accsA[0:3]), (1, accsB[0:3])):
                    i = 2 * j + t
                    seg = bufsets[k][0][pl.ds(i * 16, 16)]
                    scat(acc3, seg, k, i)
                return 0
            lax.fori_loop(0, NB // 32, node_pair, 0, unroll=4)

        th.wait()
        esrc = (dst_hbm, maeo_hbm, mseo_hbm, numo_hbm)
        ebase = wid * EDGE_PER
        pend = start(esrc, ebase, EBLK, 0)
        for b in range(nblk):
            k = b % 2
            for h in pend:
                h.wait()
            if b + 1 < nblk:
                pend = start(esrc, ebase + (b + 1) * EBLK, EBLK, (b + 1) % 2)

            def edge_pair(j, _, k=k):
                for t, acc3 in ((0, accsA[3:6]), (1, accsB[3:6])):
                    i = 2 * j + t
                    idx = bufsets[k][0][pl.ds(i * 16, 16)]
                    seg = plsc.load_gather(table_v, [idx])
                    scat(acc3, seg, k, i)
                return 0
            lax.fori_loop(0, EBLK // 32, edge_pair, 0, unroll=4)

            i_tail = EBLK // 16 - 1
            idx = bufsets[k][0][pl.ds(i_tail * 16, 16)]
            seg = plsc.load_gather(table_v, [idx])
            scat(accsA[3:6], seg, k, i_tail)

        def merge_step(i, _):
            for aA, aB in zip(accsA, accsB):
                aA[pl.ds(i * 16, 16)] = (aA[pl.ds(i * 16, 16)]
                                         + aB[pl.ds(i * 16, 16)])
            return 0
        lax.fori_loop(0, B // 16, merge_step, 0)

        for q, a in enumerate(accsA):
            pltpu.sync_copy(a, out_hbm.at[wid, q])

    return sc_kernel(batch_pad, maed, msed, numd, dst, maeo, mseo, numo)


def _combine(partials, bi_last):
    def body(p_ref, bi_ref, out_ref):
        s = jnp.sum(p_ref[...], axis=0)
        maed, msed, numd = s[0:1], s[1:2], s[2:3]
        maeo, mseo, numo = s[3:4], s[4:5], s[5:6]
        nseg_i = bi_ref[0, 0] + 1
        msk = lax.broadcasted_iota(jnp.int32, (1, B), 1) < nseg_i
        nseg = nseg_i.astype(jnp.float32)
        num = numd + numo
        batch_mae = jnp.sum(jnp.where(msk, (maed + maeo) / num, 0.0)) / nseg
        batch_mse = jnp.sum(jnp.where(msk, (msed + mseo) / num, 0.0)) / nseg
        batch_loss = batch_mae + jnp.sqrt(batch_mse)
        diag_mae = jnp.sum(jnp.where(msk, maed / numd, 0.0)) / nseg
        off_mae = jnp.sum(jnp.where(msk, maeo / numo, 0.0)) / nseg
        lane = lax.broadcasted_iota(jnp.int32, (1, 128), 1)
        out_ref[...] = (jnp.where(lane == 0, batch_loss, 0.0)
                        + jnp.where(lane == 1, batch_mae, 0.0)
                        + jnp.where(lane == 2, diag_mae, 0.0)
                        + jnp.where(lane == 3, off_mae, 0.0))

    return pl.pallas_call(
        body,
        in_specs=[pl.BlockSpec(memory_space=pltpu.VMEM),
                  pl.BlockSpec(memory_space=pltpu.SMEM)],
        out_specs=pl.BlockSpec(memory_space=pltpu.VMEM),
        out_shape=jax.ShapeDtypeStruct((1, 128), jnp.float32),
    )(partials, bi_last)


def kernel(pred_diag, pred_off_diag, real_diag, real_off_diag,
           mask_diag, mask_off_diag, batch_index, dst_index):
    t = lambda x: x.transpose(1, 2, 0)
    maed, msed, numd, maeo, mseo, numo = _lane_triples(
        t(pred_diag), t(real_diag), t(mask_diag),
        t(pred_off_diag), t(real_off_diag), t(mask_off_diag))

    batch_pad = jnp.pad(batch_index.astype(jnp.int32), (0, NODE_TOT - N))

    partials = _segment_partials(batch_pad, maed, msed, numd,
                                 dst_index.astype(jnp.int32),
                                 maeo, mseo, numo)

    bi_last = batch_index[N - 1:].astype(jnp.int32).reshape(1, 1)
    out = _combine(partials, bi_last)
    return (out[0, 0], out[0, 1], out[0, 2], out[0, 3])


# speedup vs baseline: 1.1690x; 1.0366x over previous
import functools

import jax
import jax.numpy as jnp
from jax import lax
from jax.experimental import pallas as pl
from jax.experimental.pallas import tpu as pltpu
from jax.experimental.pallas import tpu_sc as plsc

N = 100000
E = 1600000
B = 512
NW = 32
NODE_PER = 3136
NODE_TOT = 100352
EDGE_PER = E // NW
EBLK = 2000
CL = 49152


GD = (NODE_TOT + CL - 1) // CL
GE = (E + CL - 1) // CL


def _lane_triples(pd, rd, md, po, ro, mo):
    def emit(p_ref, r_ref, m_ref, base, n_valid, outs):
        d = p_ref[...].reshape(16, CL) - r_ref[...].reshape(16, CL)
        mm = m_ref[...].reshape(16, CL)
        a = jnp.abs(d)
        t = a * mm
        ok = base + lax.broadcasted_iota(jnp.int32, (1, CL), 1) < n_valid
        ones = jnp.full((1, 16), 1.0, jnp.float32)
        red = lambda x: jnp.dot(ones, x, preferred_element_type=jnp.float32)
        vals = (red(t), red(t * a), red(mm))
        for o_ref, v in zip(outs, vals):
            o_ref[...] = jnp.where(ok, v, 0.0).reshape(CL)

    def body(pd_ref, rd_ref, md_ref, po_ref, ro_ref, mo_ref,
             maed_ref, msed_ref, numd_ref, maeo_ref, mseo_ref, numo_ref):
        i = pl.program_id(0)

        @pl.when(i < GD)
        def _():
            emit(pd_ref, rd_ref, md_ref, i * CL, N,
                 (maed_ref, msed_ref, numd_ref))

        @pl.when(i >= GD)
        def _():
            emit(po_ref, ro_ref, mo_ref, (i - GD) * CL, E,
                 (maeo_ref, mseo_ref, numo_ref))

    dspec = pl.BlockSpec((4, 4, CL), lambda i: (0, 0, jnp.minimum(i, GD - 1)))
    ospec = pl.BlockSpec((4, 4, CL), lambda i: (0, 0, jnp.maximum(i - GD, 0)))
    dout = pl.BlockSpec((CL,), lambda i: (jnp.minimum(i, GD - 1),))
    oout = pl.BlockSpec((CL,), lambda i: (jnp.maximum(i - GD, 0),))
    return pl.pallas_call(
        body,
        grid=(GD + GE,),
        in_specs=[dspec] * 3 + [ospec] * 3,
        out_specs=[dout] * 3 + [oout] * 3,
        out_shape=[jax.ShapeDtypeStruct((NODE_TOT,), jnp.float32)] * 3
        + [jax.ShapeDtypeStruct((E,), jnp.float32)] * 3,
    )(pd, rd, md, po, ro, mo)


def _segment_partials(batch_pad, maed, msed, numd, dst, maeo, mseo, numo):
    mesh = plsc.VectorSubcoreMesh(core_axis_name="c", subcore_axis_name="s")
    nblk = EDGE_PER // EBLK
    NB = NODE_PER // 2

    @functools.partial(
        pl.kernel,
        mesh=mesh,
        compiler_params=pltpu.CompilerParams(needs_layout_passes=False),
        out_type=jax.ShapeDtypeStruct((NW, 6, 512), jnp.float32),
        scratch_types=[
            pltpu.VMEM((N,), jnp.int32),
            pltpu.VMEM((EBLK,), jnp.int32),
            pltpu.VMEM((EBLK,), jnp.float32),
            pltpu.VMEM((EBLK,), jnp.float32),
            pltpu.VMEM((EBLK,), jnp.float32),
            pltpu.VMEM((EBLK,), jnp.int32),
            pltpu.VMEM((EBLK,), jnp.float32),
            pltpu.VMEM((EBLK,), jnp.float32),
            pltpu.VMEM((EBLK,), jnp.float32),
            pltpu.SemaphoreType.DMA,
            pltpu.SemaphoreType.DMA,
            pltpu.SemaphoreType.DMA,
        ] + [pltpu.VMEM((B,), jnp.float32)] * 12,
    )
    def sc_kernel(batch_hbm, maed_hbm, msed_hbm, numd_hbm,
                  dst_hbm, maeo_hbm, mseo_hbm, numo_hbm, out_hbm,
                  table_v, i0_v, a0_v, b0_v, c0_v, i1_v, a1_v, b1_v, c1_v,
                  sem0, sem1, semt,
                  d0A, d1A, d2A, o0A, o1A, o2A,
                  d0B, d1B, d2B, o0B, o1B, o2B):
        wid = lax.axis_index("s") * 2 + lax.axis_index("c")
        accsA = (d0A, d1A, d2A, o0A, o1A, o2A)
        accsB = (d0B, d1B, d2B, o0B, o1B, o2B)
        sems = (sem0, sem1)
        bufsets = ((i0_v, a0_v, b0_v, c0_v), (i1_v, a1_v, b1_v, c1_v))

        th = pltpu.make_async_copy(batch_hbm.at[pl.ds(0, N)], table_v, semt)
        th.start()

        def zero_step(i, _):
            z = jnp.zeros((16,), jnp.float32)
            for a in accsA + accsB:
                a[pl.ds(i * 16, 16)] = z
            return 0
        lax.fori_loop(0, B // 16, zero_step, 0)

        def start(srcs, off, ln, k):
            hs = []
            for hb, v in zip(srcs, bufsets[k]):
                h = pltpu.make_async_copy(hb.at[pl.ds(off, ln)],
                                          v.at[pl.ds(0, ln)], sems[k])
                h.start()
                hs.append(h)
            return hs

        def scat(acc3, seg, k, i):
            for a, v in zip(acc3, bufsets[k][1:]):
                plsc.addupdate_scatter(a, [seg], v[pl.ds(i * 16, 16)])

        nsrc = (batch_hbm, maed_hbm, msed_hbm, numd_hbm)
        nbase = wid * NODE_PER
        h0 = start(nsrc, nbase, NB, 0)
        h1 = start(nsrc, nbase + NB, NB, 1)
        for k, hs in ((0, h0), (1, h1)):
            for h in hs:
                h.wait()

            def node_pair(j, _, k=k):
                for t, acc3 in ((0, accsA[0:3]), (1, accsB[0:3])):
                    i = 2 * j + t
                    seg = bufsets[k][0][pl.ds(i * 16, 16)]
                    scat(acc3, seg, k, i)
                return 0
            lax.fori_loop(0, NB // 32, node_pair, 0, unroll=4)

        th.wait()
        esrc = (dst_hbm, maeo_hbm, mseo_hbm, numo_hbm)
        ebase = wid * EDGE_PER
        pend = start(esrc, ebase, EBLK, 0)
        for b in range(nblk):
            k = b % 2
            for h in pend:
                h.wait()
            if b + 1 < nblk:
                pend = start(esrc, ebase + (b + 1) * EBLK, EBLK, (b + 1) % 2)

            def edge_pair(j, _, k=k):
                for t, acc3 in ((0, accsA[3:6]), (1, accsB[3:6])):
                    i = 2 * j + t
                    idx = bufsets[k][0][pl.ds(i * 16, 16)]
                    seg = plsc.load_gather(table_v, [idx])
                    scat(acc3, seg, k, i)
                return 0
            lax.fori_loop(0, EBLK // 32, edge_pair, 0, unroll=4)

            i_tail = EBLK // 16 - 1
            idx = bufsets[k][0][pl.ds(i_tail * 16, 16)]
            seg = plsc.load_gather(table_v, [idx])
            scat(accsA[3:6], seg, k, i_tail)

        def merge_step(i, _):
            for aA, aB in zip(accsA, accsB):
                aA[pl.ds(i * 16, 16)] = (aA[pl.ds(i * 16, 16)]
                                         + aB[pl.ds(i * 16, 16)])
            return 0
        lax.fori_loop(0, B // 16, merge_step, 0)

        for q, a in enumerate(accsA):
            pltpu.sync_copy(a, out_hbm.at[wid, q])

    return sc_kernel(batch_pad, maed, msed, numd, dst, maeo, mseo, numo)


def _combine(partials, bi_last):
    def body(p_ref, bi_ref, out_ref):
        s = jnp.sum(p_ref[...], axis=0)
        maed, msed, numd = s[0:1], s[1:2], s[2:3]
        maeo, mseo, numo = s[3:4], s[4:5], s[5:6]
        nseg_i = bi_ref[0, 0] + 1
        msk = lax.broadcasted_iota(jnp.int32, (1, B), 1) < nseg_i
        nseg = nseg_i.astype(jnp.float32)
        num = numd + numo
        batch_mae = jnp.sum(jnp.where(msk, (maed + maeo) / num, 0.0)) / nseg
        batch_mse = jnp.sum(jnp.where(msk, (msed + mseo) / num, 0.0)) / nseg
        batch_loss = batch_mae + jnp.sqrt(batch_mse)
        diag_mae = jnp.sum(jnp.where(msk, maed / numd, 0.0)) / nseg
        off_mae = jnp.sum(jnp.where(msk, maeo / numo, 0.0)) / nseg
        lane = lax.broadcasted_iota(jnp.int32, (1, 128), 1)
        out_ref[...] = (jnp.where(lane == 0, batch_loss, 0.0)
                        + jnp.where(lane == 1, batch_mae, 0.0)
                        + jnp.where(lane == 2, diag_mae, 0.0)
                        + jnp.where(lane == 3, off_mae, 0.0))

    return pl.pallas_call(
        body,
        in_specs=[pl.BlockSpec(memory_space=pltpu.VMEM),
                  pl.BlockSpec(memory_space=pltpu.SMEM)],
        out_specs=pl.BlockSpec(memory_space=pltpu.VMEM),
        out_shape=jax.ShapeDtypeStruct((1, 128), jnp.float32),
    )(partials, bi_last)


def kernel(pred_diag, pred_off_diag, real_diag, real_off_diag,
           mask_diag, mask_off_diag, batch_index, dst_index):
    t = lambda x: x.transpose(1, 2, 0)
    maed, msed, numd, maeo, mseo, numo = _lane_triples(
        t(pred_diag), t(real_diag), t(mask_diag),
        t(pred_off_diag), t(real_off_diag), t(mask_off_diag))

    batch_pad = jnp.pad(batch_index.astype(jnp.int32), (0, NODE_TOT - N))

    partials = _segment_partials(batch_pad, maed, msed, numd,
                                 dst_index.astype(jnp.int32),
                                 maeo, mseo, numo)

    bi_last = batch_index[N - 1:].astype(jnp.int32).reshape(1, 1)
    out = _combine(partials, bi_last)
    return (out[0, 0], out[0, 1], out[0, 2], out[0, 3])
